# Initial kernel scaffold; baseline (speedup 1.0000x reference)
import functools

import jax
import jax.numpy as jnp
from jax import lax
from jax.experimental import pallas as pl
from jax.experimental.pallas import tpu as pltpu
from jax.experimental.pallas import tpu_sc as plsc

N_NODES = 10000
N_EDGES = 320000
D_IN = 128
D_OUT = 64

NC, NS, L = 2, 16, 16
NW = NC * NS
CHUNK = 128
CPT = -(-N_EDGES // (NW * CHUNK))
E_PAD = NW * CPT * CHUNK
ROW_BLK = 512
N_PAD = ROW_BLK * (-(-(N_NODES + 1) // ROW_BLK))


def _deg_body(dst_hbm, z_hbm, degp_hbm, idx_v, ones_v, deg_sh):
    c = lax.axis_index("c")
    s = lax.axis_index("s")
    wid = c * NS + s

    @pl.when(s == 0)
    def _():
        pltpu.sync_copy(z_hbm, deg_sh)

    for i in range(CHUNK // L):
        ones_v[pl.ds(i * L, L)] = jnp.ones((L,), jnp.float32)
    plsc.subcore_barrier()

    base = wid * (CPT * CHUNK)

    def body(j, carry):
        pltpu.sync_copy(dst_hbm.at[pl.ds(base + j * CHUNK, CHUNK)], idx_v)
        pltpu.sync_copy(ones_v, deg_sh.at[idx_v], add=True)
        return carry

    lax.fori_loop(0, CPT, body, 0)
    plsc.subcore_barrier()

    @pl.when(s == 0)
    def _():
        pltpu.sync_copy(deg_sh, degp_hbm.at[c])


def _scatter_body(src_hbm, dst_hbm, h2_hbm, z_hbm, accp_hbm,
                  src_v, dst_v, rows_v, acc_sh, gsem):
    c = lax.axis_index("c")
    s = lax.axis_index("s")
    wid = c * NS + s

    @pl.when(jnp.logical_and(s == 0, c == 0))
    def _():
        pltpu.sync_copy(h2_hbm, acc_sh)

    @pl.when(jnp.logical_and(s == 0, c == 1))
    def _():
        pltpu.sync_copy(z_hbm, acc_sh)

    plsc.subcore_barrier()

    base = wid * (CPT * CHUNK)

    def body(j, carry):
        off = base + j * CHUNK
        pltpu.sync_copy(src_hbm.at[pl.ds(off, CHUNK)], src_v)
        pltpu.sync_copy(dst_hbm.at[pl.ds(off, CHUNK)], dst_v)
        pltpu.async_copy(h2_hbm.at[src_v], rows_v, gsem).wait()
        pltpu.sync_copy(rows_v, acc_sh.at[dst_v], add=True)
        return carry

    lax.fori_loop(0, CPT, body, 0)
    plsc.subcore_barrier()

    @pl.when(s == 0)
    def _():
        pltpu.sync_copy(acc_sh, accp_hbm.at[c])


def _h2_tc_body(x_ref, w_ref, degp_ref, h2_ref, dis_ref):
    deg = degp_ref[0, :] + degp_ref[1, :] + 1.0
    dis = lax.rsqrt(deg)
    h = jnp.dot(x_ref[...], w_ref[...], preferred_element_type=jnp.float32)
    h2_ref[...] = h * dis[:, None]
    dis_ref[...] = dis[:, None]


def _final_tc_body(accp_ref, dis_ref, b_ref, out_ref):
    a = accp_ref[0] + accp_ref[1]
    out_ref[...] = a * dis_ref[...] + b_ref[...]


def kernel(x, edge_index, W, b):
    src = edge_index[0].astype(jnp.int32)
    dst = edge_index[1].astype(jnp.int32)
    pad = E_PAD - N_EDGES
    src_p = jnp.concatenate([src, jnp.zeros((pad,), jnp.int32)])
    dst_p = jnp.concatenate([dst, jnp.full((pad,), N_NODES, jnp.int32)])
    x_p = jnp.pad(x, ((0, N_PAD - N_NODES), (0, 0)))
    z_row = jnp.zeros((N_PAD,), jnp.float32)
    z_mat = jnp.zeros((N_PAD, D_OUT), jnp.float32)

    mesh = plsc.VectorSubcoreMesh(core_axis_name="c", subcore_axis_name="s")

    deg_k = functools.partial(
        pl.kernel,
        out_type=jax.ShapeDtypeStruct((NC, N_PAD), jnp.float32),
        mesh=mesh,
        scratch_types=[
            pltpu.VMEM((CHUNK,), jnp.int32),
            pltpu.VMEM((CHUNK,), jnp.float32),
            pltpu.VMEM_SHARED((N_PAD,), jnp.float32),
        ],
    )(_deg_body)
    degp = deg_k(dst_p, z_row)

    n_blocks = N_PAD // ROW_BLK
    h2, dis = pl.pallas_call(
        _h2_tc_body,
        grid=(n_blocks,),
        in_specs=[
            pl.BlockSpec((ROW_BLK, D_IN), lambda i: (i, 0)),
            pl.BlockSpec((D_IN, D_OUT), lambda i: (0, 0)),
            pl.BlockSpec((NC, ROW_BLK), lambda i: (0, i)),
        ],
        out_specs=[
            pl.BlockSpec((ROW_BLK, D_OUT), lambda i: (i, 0)),
            pl.BlockSpec((ROW_BLK, 1), lambda i: (i, 0)),
        ],
        out_shape=[
            jax.ShapeDtypeStruct((N_PAD, D_OUT), jnp.float32),
            jax.ShapeDtypeStruct((N_PAD, 1), jnp.float32),
        ],
    )(x_p, W, degp)

    scat_k = functools.partial(
        pl.kernel,
        out_type=jax.ShapeDtypeStruct((NC, N_PAD, D_OUT), jnp.float32),
        mesh=mesh,
        scratch_types=[
            pltpu.VMEM((CHUNK,), jnp.int32),
            pltpu.VMEM((CHUNK,), jnp.int32),
            pltpu.VMEM((CHUNK, D_OUT), jnp.float32),
            pltpu.VMEM_SHARED((N_PAD, D_OUT), jnp.float32),
            pltpu.SemaphoreType.DMA,
        ],
    )(_scatter_body)
    accp = scat_k(src_p, dst_p, h2, z_mat)

    out = pl.pallas_call(
        _final_tc_body,
        grid=(n_blocks,),
        in_specs=[
            pl.BlockSpec((NC, ROW_BLK, D_OUT), lambda i: (0, i, 0)),
            pl.BlockSpec((ROW_BLK, 1), lambda i: (i, 0)),
            pl.BlockSpec((1, D_OUT), lambda i: (0, 0)),
        ],
        out_specs=pl.BlockSpec((ROW_BLK, D_OUT), lambda i: (i, 0)),
        out_shape=jax.ShapeDtypeStruct((N_PAD, D_OUT), jnp.float32),
    )(accp, dis, b.reshape(1, D_OUT))

    return out[:N_NODES]


# speedup vs baseline: 20.9530x; 20.9530x over previous
import functools

import jax
import jax.numpy as jnp
from jax import lax
from jax.experimental import pallas as pl
from jax.experimental.pallas import tpu as pltpu
from jax.experimental.pallas import tpu_sc as plsc

N_NODES = 10000
N_EDGES = 320000
D_IN = 128
D_OUT = 64

NC, NS, L = 2, 16, 16
NW = NC * NS
CHUNK = 128
CPT = -(-N_EDGES // (NW * CHUNK))
E_PAD = NW * CPT * CHUNK
ROW_BLK = 512
N_PAD = ROW_BLK * (-(-(N_NODES + 1) // ROW_BLK))


def _deg_body(dst_hbm, z_hbm, degp_hbm, idx_v, ones_v, deg_sh):
    c = lax.axis_index("c")
    s = lax.axis_index("s")
    wid = c * NS + s

    @pl.when(s == 0)
    def _():
        pltpu.sync_copy(z_hbm, deg_sh)

    for i in range(CHUNK // L):
        ones_v[pl.ds(i * L, L)] = jnp.ones((L,), jnp.float32)
    plsc.subcore_barrier()

    base = wid * (CPT * CHUNK)

    def body(j, carry):
        pltpu.sync_copy(dst_hbm.at[pl.ds(base + j * CHUNK, CHUNK)], idx_v)
        pltpu.sync_copy(ones_v, deg_sh.at[idx_v], add=True)
        return carry

    lax.fori_loop(0, CPT, body, 0)
    plsc.subcore_barrier()

    @pl.when(s == 0)
    def _():
        pltpu.sync_copy(deg_sh, degp_hbm.at[c])


def _scatter_body(src_hbm, dst_hbm, h2_hbm, z_hbm, accp_hbm,
                  src_v, dst_v, rows_v, acc_sh, gsem):
    c = lax.axis_index("c")
    s = lax.axis_index("s")
    wid = c * NS + s

    @pl.when(jnp.logical_and(s == 0, c == 0))
    def _():
        pltpu.sync_copy(h2_hbm, acc_sh)

    @pl.when(jnp.logical_and(s == 0, c == 1))
    def _():
        pltpu.sync_copy(z_hbm, acc_sh)

    plsc.subcore_barrier()

    base = wid * (CPT * CHUNK)

    def body(j, carry):
        off = base + j * CHUNK
        pltpu.sync_copy(src_hbm.at[pl.ds(off, CHUNK)], src_v)
        pltpu.sync_copy(dst_hbm.at[pl.ds(off, CHUNK)], dst_v)
        pltpu.async_copy(h2_hbm.at[src_v], rows_v, gsem).wait()
        pltpu.sync_copy(rows_v, acc_sh.at[dst_v], add=True)
        return carry

    lax.fori_loop(0, CPT, body, 0)
    plsc.subcore_barrier()

    @pl.when(s == 0)
    def _():
        pltpu.sync_copy(acc_sh, accp_hbm.at[c])


def _h2_tc_body(x_ref, w_ref, degp_ref, h2_ref, dis_ref):
    deg = degp_ref[0, :] + degp_ref[1, :] + 1.0
    dis = lax.rsqrt(deg)
    h = jnp.dot(x_ref[...], w_ref[...], preferred_element_type=jnp.float32)
    h2_ref[...] = h * dis[:, None]
    dis_ref[...] = dis[:, None]


def _final_tc_body(accp_ref, dis_ref, b_ref, out_ref):
    a = accp_ref[0] + accp_ref[1]
    out_ref[...] = a * dis_ref[...] + b_ref[...]


def kernel(x, edge_index, W, b):
    src = edge_index[0].astype(jnp.int32)
    dst = edge_index[1].astype(jnp.int32)
    pad = E_PAD - N_EDGES
    src_p = jnp.concatenate([src, jnp.zeros((pad,), jnp.int32)])
    dst_p = jnp.concatenate([dst, jnp.full((pad,), N_NODES, jnp.int32)])
    x_p = jnp.pad(x, ((0, N_PAD - N_NODES), (0, 0)))
    z_row = jnp.zeros((N_PAD,), jnp.float32)
    z_mat = jnp.zeros((N_PAD, D_OUT), jnp.float32)

    mesh = plsc.VectorSubcoreMesh(core_axis_name="c", subcore_axis_name="s")

    deg_k = functools.partial(
        pl.kernel,
        out_type=jax.ShapeDtypeStruct((NC, N_PAD), jnp.float32),
        mesh=mesh,
        scratch_types=[
            pltpu.VMEM((CHUNK,), jnp.int32),
            pltpu.VMEM((CHUNK,), jnp.float32),
            pltpu.VMEM_SHARED((N_PAD,), jnp.float32),
        ],
    )(_deg_body)
    degp = deg_k(dst_p, z_row)

    n_blocks = N_PAD // ROW_BLK
    h2, dis = pl.pallas_call(
        _h2_tc_body,
        grid=(n_blocks,),
        in_specs=[
            pl.BlockSpec((ROW_BLK, D_IN), lambda i: (i, 0)),
            pl.BlockSpec((D_IN, D_OUT), lambda i: (0, 0)),
            pl.BlockSpec((NC, ROW_BLK), lambda i: (0, i)),
        ],
        out_specs=[
            pl.BlockSpec((ROW_BLK, D_OUT), lambda i: (i, 0)),
            pl.BlockSpec((ROW_BLK, 1), lambda i: (i, 0)),
        ],
        out_shape=[
            jax.ShapeDtypeStruct((N_PAD, D_OUT), jnp.float32),
            jax.ShapeDtypeStruct((N_PAD, 1), jnp.float32),
        ],
    )(x_p, W, degp)

    scat_k = functools.partial(
        pl.kernel,
        out_type=jax.ShapeDtypeStruct((NC, N_PAD, D_OUT), jnp.float32),
        mesh=mesh,
        compiler_params=pltpu.CompilerParams(use_tc_tiling_on_sc=False),
        scratch_types=[
            pltpu.VMEM((CHUNK,), jnp.int32),
            pltpu.VMEM((CHUNK,), jnp.int32),
            pltpu.VMEM((CHUNK, D_OUT), jnp.float32),
            pltpu.VMEM_SHARED((N_PAD, D_OUT), jnp.float32),
            pltpu.SemaphoreType.DMA,
        ],
    )(_scatter_body)
    accp = scat_k(src_p, dst_p, h2, z_mat)

    out = pl.pallas_call(
        _final_tc_body,
        grid=(n_blocks,),
        in_specs=[
            pl.BlockSpec((NC, ROW_BLK, D_OUT), lambda i: (0, i, 0)),
            pl.BlockSpec((ROW_BLK, 1), lambda i: (i, 0)),
            pl.BlockSpec((1, D_OUT), lambda i: (0, 0)),
        ],
        out_specs=pl.BlockSpec((ROW_BLK, D_OUT), lambda i: (i, 0)),
        out_shape=jax.ShapeDtypeStruct((N_PAD, D_OUT), jnp.float32),
    )(accp, dis, b.reshape(1, D_OUT))

    return out[:N_NODES]
